# Initial kernel scaffold; baseline (speedup 1.0000x reference)
import jax
import jax.numpy as jnp
from jax.experimental import pallas as pl


def kernel(pred, target, feats, A_row_ptr, A_col_ind, A_vals, A_row_idx, b, edge_index, epoch):
    raise NotImplementedError("write your pallas kernel here")


# speedup vs baseline: 30.9681x; 30.9681x over previous
import functools

import jax
import jax.numpy as jnp
from jax import lax
from jax.experimental import pallas as pl
from jax.experimental.pallas import tpu as pltpu
from jax.experimental.pallas import tpu_sc as plsc

_N = 10000
_NP = 10240
_NT = 16
_SL = _NP // _NT
_E = 320000
_EPT = _E // _NT
_NNZ = 320000
_ZPT = _NNZ // _NT
_GE = _EPT // 16
_GZ = _ZPT // 16
_GS = _SL // 16
_D = 128


def _sqrt16(s):
    bits = plsc.bitcast(s, jnp.int32)
    r = plsc.bitcast(jnp.int32(0x5F3759DF) - (bits >> 1), jnp.float32)
    for _ in range(3):
        r = r * (1.5 - 0.5 * s * r * r)
    return jnp.where(s > 0.0, s * r, 0.0)


def _scatter_max(acc_ref, idx, val):
    def cond(act):
        return jnp.any(act)

    def body(act):
        cur = plsc.load_gather(acc_ref, [idx], mask=act)
        win = act & (val > cur)
        plsc.store_scatter(acc_ref, [idx], val, mask=win)
        cur2 = plsc.load_gather(acc_ref, [idx], mask=win)
        return win & (cur2 < val)

    lax.while_loop(cond, body, jnp.ones((16,), jnp.bool_))


def _kernel_body(pred_hbm, target_hbm, featsf_hbm, colind_hbm, vals_hbm,
                 rowidx_hbm, b_hbm, edge_hbm, out_hbm,
                 pred_ref, gbuf_ref, acc_ref, b1_ref, b2_ref, b3_ref,
                 fbuf_ref, mbuf_ref, m16_ref, mb16_ref, vec16_ref,
                 vecb16_ref, grad_ref, w_ref, out_sl_ref, res_ref, bsl_ref,
                 tsl_ref, outv_ref, sh_arena):
    _PART = 0
    _FULL = _NT * _NP
    _VA = _FULL + _NP
    _VB = _VA + 16 * _NT
    sid = lax.axis_index("s")
    base = sid * _SL
    ebase = sid * _EPT
    zbase = sid * _ZPT
    zero16 = jnp.zeros((16,), jnp.float32)

    def zero_acc():
        def zb(i, _):
            acc_ref[pl.ds(pl.multiple_of(i * 16, 16), 16)] = zero16
            return 0
        lax.fori_loop(0, _NP // 16, zb, 0)

    def merge_slices(op):
        plsc.subcore_barrier()
        pltpu.sync_copy(acc_ref, sh_arena.at[pl.ds(sid * _NP, _NP)])
        plsc.subcore_barrier()
        for j in range(_NT):
            pltpu.sync_copy(sh_arena.at[pl.ds(j * _NP + base, _SL)],
                            mbuf_ref.at[j])

        def mb(gi, _):
            o = pl.multiple_of(gi * 16, 16)
            m = mbuf_ref[0, pl.ds(o, 16)]
            for j in range(1, _NT):
                m = op(m, mbuf_ref[j, pl.ds(o, 16)])
            res = m
            return res

        return mb

    def merge_to(dst_ref, op):
        mb = merge_slices(op)

        def body(gi, _):
            o = pl.multiple_of(gi * 16, 16)
            dst_ref[pl.ds(o, 16)] = mb(gi, None)
            return 0
        lax.fori_loop(0, _GS, body, 0)

    def broadcast_full(slice_ref, dst_ref):
        plsc.subcore_barrier()
        pltpu.sync_copy(slice_ref, sh_arena.at[pl.ds(_FULL + base, _SL)])
        plsc.subcore_barrier()
        pltpu.sync_copy(sh_arena.at[pl.ds(_FULL, _NP)], dst_ref)

    def edge_pass(absdiff):
        def body(g, _):
            o = pl.multiple_of(g * 16, 16)
            s = b1_ref[pl.ds(o, 16)]
            d = b2_ref[pl.ds(o, 16)]
            if absdiff:
                ps = plsc.load_gather(pred_ref, [s])
                pd = plsc.load_gather(pred_ref, [d])
                v = jnp.abs(pd - ps)
            else:
                v = plsc.load_gather(gbuf_ref, [s])
            _scatter_max(acc_ref, d, v)
            return 0
        lax.fori_loop(0, _GE, body, 0)

    pltpu.sync_copy(pred_hbm, pred_ref)
    pltpu.sync_copy(edge_hbm.at[pl.ds(ebase, _EPT)], b1_ref)
    pltpu.sync_copy(edge_hbm.at[pl.ds(_E + ebase, _EPT)], b2_ref)
    pltpu.sync_copy(target_hbm.at[pl.ds(base, _SL)], tsl_ref)
    pltpu.sync_copy(b_hbm.at[pl.ds(base, _SL)], bsl_ref)

    zero_acc()
    edge_pass(absdiff=True)
    merge_to(grad_ref, jnp.maximum)

    def gmb(gi, m):
        return jnp.maximum(m, grad_ref[pl.ds(pl.multiple_of(gi * 16, 16), 16)])
    vm = lax.fori_loop(0, _GS, gmb, zero16)
    vec16_ref[...] = vm
    plsc.subcore_barrier()
    pltpu.sync_copy(vec16_ref, sh_arena.at[pl.ds(_VA + sid * 16, 16)])
    plsc.subcore_barrier()
    pltpu.sync_copy(sh_arena.at[pl.ds(_VA, 16 * _NT)], m16_ref)
    m = m16_ref[pl.ds(0, 16)]
    for j in range(1, _NT):
        m = jnp.maximum(m, m16_ref[pl.ds(j * 16, 16)])
    gmax = jnp.max(m)
    inv_g = 1.0 / (jnp.full((16,), gmax, jnp.float32) + 1e-8)

    idx0 = lax.iota(jnp.int32, 16) * _D

    def qb(kc, _):
        o = pl.multiple_of(kc * 16, 16)
        grow = base + o
        gn = grad_ref[pl.ds(o, 16)] * inv_g

        @pl.when(grow < _N)
        def _():
            pltpu.sync_copy(featsf_hbm.at[pl.ds(grow * _D, 16 * _D)], fbuf_ref)

            def cb(ci, accv):
                cbase = ci * 16
                for j in range(16):
                    v = plsc.load_gather(fbuf_ref, [idx0 + (cbase + j)])
                    accv = accv + v * v
                return accv
            ss = lax.fori_loop(0, _D // 16, cb, zero16)
            q = 1.0 / (1.0 + _sqrt16(ss))
            w_ref[pl.ds(o, 16)] = 1.0 + gn * q

        @pl.when(grow >= _N)
        def _():
            w_ref[pl.ds(o, 16)] = 1.0 + gn
        return 0
    lax.fori_loop(0, _GS, qb, 0)

    broadcast_full(w_ref, gbuf_ref)

    zero_acc()
    edge_pass(absdiff=False)
    merge_to(grad_ref, jnp.maximum)

    def h1(gi, _):
        o = pl.multiple_of(gi * 16, 16)
        out_sl_ref[pl.ds(o, 16)] = jnp.maximum(w_ref[pl.ds(o, 16)],
                                               0.5 * grad_ref[pl.ds(o, 16)])
        return 0
    lax.fori_loop(0, _GS, h1, 0)

    broadcast_full(grad_ref, gbuf_ref)

    zero_acc()
    edge_pass(absdiff=False)
    merge_to(grad_ref, jnp.maximum)

    def h2(gi, _):
        o = pl.multiple_of(gi * 16, 16)
        out_sl_ref[pl.ds(o, 16)] = jnp.maximum(out_sl_ref[pl.ds(o, 16)],
                                               0.25 * grad_ref[pl.ds(o, 16)])
        return 0
    lax.fori_loop(0, _GS, h2, 0)

    pltpu.sync_copy(colind_hbm.at[pl.ds(zbase, _ZPT)], b1_ref)
    pltpu.sync_copy(rowidx_hbm.at[pl.ds(zbase, _ZPT)], b2_ref)
    pltpu.sync_copy(vals_hbm.at[pl.ds(zbase, _ZPT)], b3_ref)
    zero_acc()

    def zb(g, _):
        o = pl.multiple_of(g * 16, 16)
        c = b1_ref[pl.ds(o, 16)]
        r = b2_ref[pl.ds(o, 16)]
        v = b3_ref[pl.ds(o, 16)]
        pc = plsc.load_gather(pred_ref, [c])
        plsc.addupdate_scatter(acc_ref, [r], v * pc)
        return 0
    lax.fori_loop(0, _GZ, zb, 0)
    merge_to(res_ref, jnp.add)

    def lpb(gi, v):
        o = pl.multiple_of(gi * 16, 16)
        r = res_ref[pl.ds(o, 16)] - bsl_ref[pl.ds(o, 16)]
        return v + out_sl_ref[pl.ds(o, 16)] * r * r
    lp_vec = lax.fori_loop(0, _GS, lpb, zero16)

    def ldb(gi, v):
        o = pl.multiple_of(gi * 16, 16)
        p = pred_ref[pl.ds(base + o, 16)]
        t = tsl_ref[pl.ds(o, 16)]
        d = p - t
        return v + d * d
    ld_vec = lax.fori_loop(0, _GS, ldb, zero16)

    vec16_ref[...] = ld_vec
    vecb16_ref[...] = lp_vec
    plsc.subcore_barrier()
    pltpu.sync_copy(vec16_ref, sh_arena.at[pl.ds(_VA + sid * 16, 16)])
    pltpu.sync_copy(vecb16_ref, sh_arena.at[pl.ds(_VB + sid * 16, 16)])
    plsc.subcore_barrier()

    @pl.when(sid == 0)
    def _():
        pltpu.sync_copy(sh_arena.at[pl.ds(_VA, 16 * _NT)], m16_ref)
        pltpu.sync_copy(sh_arena.at[pl.ds(_VB, 16 * _NT)], mb16_ref)
        sa = m16_ref[pl.ds(0, 16)]
        sb = mb16_ref[pl.ds(0, 16)]
        for j in range(1, _NT):
            sa = sa + m16_ref[pl.ds(j * 16, 16)]
            sb = sb + mb16_ref[pl.ds(j * 16, 16)]
        total = (jnp.sum(sa) + jnp.sum(sb)) * (1.0 / _N)
        outv_ref[...] = jnp.full((16,), total, jnp.float32)
        pltpu.sync_copy(outv_ref, out_hbm)


@jax.jit
def _sc_loss(pred_p, target_p, featsf, colind, vals, rowidx, b_p, edge_index):
    mesh = plsc.VectorSubcoreMesh(core_axis_name="c", subcore_axis_name="s",
                                  num_cores=1, num_subcores=_NT)
    f32 = jnp.float32
    run = pl.kernel(
        _kernel_body,
        out_type=jax.ShapeDtypeStruct((16,), f32),
        mesh=mesh,
        compiler_params=pltpu.CompilerParams(needs_layout_passes=False),
        scratch_types=[
            pltpu.VMEM((_NP,), f32),
            pltpu.VMEM((_NP,), f32),
            pltpu.VMEM((_NP,), f32),
            pltpu.VMEM((_EPT,), jnp.int32),
            pltpu.VMEM((_EPT,), jnp.int32),
            pltpu.VMEM((_ZPT,), f32),
            pltpu.VMEM((16 * _D,), f32),
            pltpu.VMEM((_NT, _SL), f32),
            pltpu.VMEM((16 * _NT,), f32),
            pltpu.VMEM((16 * _NT,), f32),
            pltpu.VMEM((16,), f32),
            pltpu.VMEM((16,), f32),
            pltpu.VMEM((_SL,), f32),
            pltpu.VMEM((_SL,), f32),
            pltpu.VMEM((_SL,), f32),
            pltpu.VMEM((_SL,), f32),
            pltpu.VMEM((_SL,), f32),
            pltpu.VMEM((_SL,), f32),
            pltpu.VMEM((16,), f32),
            pltpu.VMEM_SHARED(((_NT + 1) * _NP + 32 * _NT,), f32),
        ],
    )
    return run(pred_p, target_p, featsf, colind, vals, rowidx, b_p, edge_index)


def kernel(pred, target, feats, A_row_ptr, A_col_ind, A_vals, A_row_idx, b,
           edge_index, epoch):
    del A_row_ptr, epoch
    padn = _NP - _N
    pred_p = jnp.pad(pred, (0, padn))
    target_p = jnp.pad(target, (0, padn))
    b_p = jnp.pad(b, (0, padn))
    featsf = feats.reshape(-1)
    out = _sc_loss(pred_p, target_p, featsf, A_col_ind, A_vals, A_row_idx,
                   b_p, edge_index.reshape(-1))
    return out[0]


# speedup vs baseline: 52.4145x; 1.6925x over previous
import functools

import jax
import jax.numpy as jnp
from jax import lax
from jax.experimental import pallas as pl
from jax.experimental.pallas import tpu as pltpu
from jax.experimental.pallas import tpu_sc as plsc

_N = 10000
_NP = 10240
_NT = 16
_SL = _NP // _NT
_E = 320000
_EPT = _E // _NT
_NNZ = 320000
_ZPT = _NNZ // _NT
_GE = _EPT // 16
_GZ = _ZPT // 16
_GS = _SL // 16
_D = 128
_EK = 10


def _sqrt16(s):
    bits = plsc.bitcast(s, jnp.int32)
    r = plsc.bitcast(jnp.int32(0x5F3759DF) - (bits >> 1), jnp.float32)
    for _ in range(3):
        r = r * (1.5 - 0.5 * s * r * r)
    return jnp.where(s > 0.0, s * r, 0.0)


def _scatter_max(acc_ref, idx, val):
    def cond(act):
        return jnp.any(act)

    def body(act):
        cur = plsc.load_gather(acc_ref, [idx], mask=act)
        win = act & (val > cur)
        plsc.store_scatter(acc_ref, [idx], val, mask=win)
        cur2 = plsc.load_gather(acc_ref, [idx], mask=win)
        return win & (cur2 < val)

    lax.while_loop(cond, body, jnp.ones((16,), jnp.bool_))


def _kernel_body(pred_hbm, target_hbm, featsf_hbm, colind_hbm, vals_hbm,
                 rowidx_hbm, b_hbm, edge_hbm, out_hbm,
                 pred_ref, gbuf_ref, acc_ref, b1_ref, b2_ref, b3_ref,
                 mbuf_ref, m16_ref, mb16_ref, vec16_ref,
                 vecb16_ref, grad_ref, w_ref, out_sl_ref, res_ref, bsl_ref,
                 tsl_ref, outv_ref, dsem, sh_arena):
    _PART = 0
    _FULL = _NT * _NP
    _VA = _FULL + _NP
    _VB = _VA + 16 * _NT
    sid = lax.axis_index("s")
    base = sid * _SL
    ebase = sid * _EPT
    zbase = sid * _ZPT
    zero16 = jnp.zeros((16,), jnp.float32)

    def zero_acc():
        def zb(i, _):
            acc_ref[pl.ds(pl.multiple_of(i * 16, 16), 16)] = zero16
            return 0
        lax.fori_loop(0, _NP // 16, zb, 0)

    def merge_slices(op):
        plsc.subcore_barrier()
        pltpu.sync_copy(acc_ref, sh_arena.at[pl.ds(sid * _NP, _NP)])
        plsc.subcore_barrier()
        descs = [
            pltpu.async_copy(sh_arena.at[pl.ds(j * _NP + base, _SL)],
                             mbuf_ref.at[j], dsem)
            for j in range(_NT)
        ]
        for dsc in descs:
            dsc.wait()

        def mb(gi, _):
            o = pl.multiple_of(gi * 16, 16)
            m = mbuf_ref[0, pl.ds(o, 16)]
            for j in range(1, _NT):
                m = op(m, mbuf_ref[j, pl.ds(o, 16)])
            res = m
            return res

        return mb

    def merge_to(dst_ref, op):
        mb = merge_slices(op)

        def body(gi, _):
            o = pl.multiple_of(gi * 16, 16)
            dst_ref[pl.ds(o, 16)] = mb(gi, None)
            return 0
        lax.fori_loop(0, _GS, body, 0)

    def broadcast_full(slice_ref, dst_ref):
        plsc.subcore_barrier()
        pltpu.sync_copy(slice_ref, sh_arena.at[pl.ds(_FULL + base, _SL)])
        plsc.subcore_barrier()
        pltpu.sync_copy(sh_arena.at[pl.ds(_FULL, _NP)], dst_ref)

    def edge_pass(absdiff):
        def group_vals(g):
            o = pl.multiple_of(g * 16, 16)
            d = b2_ref[pl.ds(o, 16)]
            if absdiff:
                s = b1_ref[pl.ds(o, 16)]
                ps = plsc.load_gather(pred_ref, [s])
                pd = plsc.load_gather(pred_ref, [d])
                v = jnp.abs(pd - ps)
            else:
                s = b1_ref[pl.ds(o, 16)]
                v = plsc.load_gather(gbuf_ref, [s])
            return d, v

        def body(sg, _):
            g0 = sg * _EK
            pend = jnp.zeros((16,), jnp.bool_)
            for k in range(_EK):
                d, v = group_vals(g0 + k)
                cur = plsc.load_gather(acc_ref, [d])
                win = v > cur
                plsc.store_scatter(acc_ref, [d], v, mask=win)
                cur2 = plsc.load_gather(acc_ref, [d], mask=win)
                pend = pend | (win & (cur2 < v))

            @pl.when(jnp.any(pend))
            def _():
                for k in range(_EK):
                    d, v = group_vals(g0 + k)
                    _scatter_max(acc_ref, d, v)
            return 0
        lax.fori_loop(0, _GE // _EK, body, 0)

    descs = [
        pltpu.async_copy(pred_hbm, pred_ref, dsem),
        pltpu.async_copy(edge_hbm.at[pl.ds(ebase, _EPT)], b1_ref, dsem),
        pltpu.async_copy(edge_hbm.at[pl.ds(_E + ebase, _EPT)], b2_ref, dsem),
        pltpu.async_copy(target_hbm.at[pl.ds(base, _SL)], tsl_ref, dsem),
        pltpu.async_copy(b_hbm.at[pl.ds(base, _SL)], bsl_ref, dsem),
    ]
    for dsc in descs:
        dsc.wait()

    zero_acc()
    edge_pass(absdiff=True)
    merge_to(grad_ref, jnp.maximum)

    def gmb(gi, m):
        return jnp.maximum(m, grad_ref[pl.ds(pl.multiple_of(gi * 16, 16), 16)])
    vm = lax.fori_loop(0, _GS, gmb, zero16)
    vec16_ref[...] = vm
    plsc.subcore_barrier()
    pltpu.sync_copy(vec16_ref, sh_arena.at[pl.ds(_VA + sid * 16, 16)])
    plsc.subcore_barrier()
    pltpu.sync_copy(sh_arena.at[pl.ds(_VA, 16 * _NT)], m16_ref)
    m = m16_ref[pl.ds(0, 16)]
    for j in range(1, _NT):
        m = jnp.maximum(m, m16_ref[pl.ds(j * 16, 16)])
    gmax = jnp.max(m)
    inv_g = 1.0 / (jnp.full((16,), gmax, jnp.float32) + 1e-8)

    idx0 = lax.iota(jnp.int32, 16) * _D

    def qchunk(kc8, _):
        crow = base + kc8 * 128

        @pl.when(crow + 128 <= _N)
        def _():
            pltpu.sync_copy(featsf_hbm.at[pl.ds(crow * _D, 128 * _D)],
                            b3_ref.at[pl.ds(0, 128 * _D)])

        @pl.when((crow + 128 > _N) & (crow < _N))
        def _():
            for j in range(8):
                @pl.when(crow + j * 16 < _N)
                def _(j=j):
                    pltpu.sync_copy(
                        featsf_hbm.at[pl.ds((crow + j * 16) * _D, 16 * _D)],
                        b3_ref.at[pl.ds(j * 16 * _D, 16 * _D)])

        for j in range(8):
            o = pl.multiple_of(kc8 * 128 + j * 16, 16)
            gn = grad_ref[pl.ds(o, 16)] * inv_g
            grow = base + o

            @pl.when(grow < _N)
            def _(j=j, o=o, gn=gn):
                def cb(ci, accv):
                    cbase = j * 16 * _D + ci * 16
                    for t in range(16):
                        v = plsc.load_gather(b3_ref, [idx0 + (cbase + t)])
                        accv = accv + v * v
                    return accv
                ss = lax.fori_loop(0, _D // 16, cb, zero16)
                q = 1.0 / (1.0 + _sqrt16(ss))
                w_ref[pl.ds(o, 16)] = 1.0 + gn * q

            @pl.when(grow >= _N)
            def _(o=o, gn=gn):
                w_ref[pl.ds(o, 16)] = 1.0 + gn
        return 0
    lax.fori_loop(0, _SL // 128, qchunk, 0)

    broadcast_full(w_ref, gbuf_ref)

    zero_acc()
    edge_pass(absdiff=False)
    merge_to(grad_ref, jnp.maximum)

    def h1(gi, _):
        o = pl.multiple_of(gi * 16, 16)
        out_sl_ref[pl.ds(o, 16)] = jnp.maximum(w_ref[pl.ds(o, 16)],
                                               0.5 * grad_ref[pl.ds(o, 16)])
        return 0
    lax.fori_loop(0, _GS, h1, 0)

    broadcast_full(grad_ref, gbuf_ref)

    zero_acc()
    edge_pass(absdiff=False)
    merge_to(grad_ref, jnp.maximum)

    def h2(gi, _):
        o = pl.multiple_of(gi * 16, 16)
        out_sl_ref[pl.ds(o, 16)] = jnp.maximum(out_sl_ref[pl.ds(o, 16)],
                                               0.25 * grad_ref[pl.ds(o, 16)])
        return 0
    lax.fori_loop(0, _GS, h2, 0)

    descs = [
        pltpu.async_copy(colind_hbm.at[pl.ds(zbase, _ZPT)], b1_ref, dsem),
        pltpu.async_copy(rowidx_hbm.at[pl.ds(zbase, _ZPT)], b2_ref, dsem),
        pltpu.async_copy(vals_hbm.at[pl.ds(zbase, _ZPT)], b3_ref, dsem),
    ]
    zero_acc()
    for dsc in descs:
        dsc.wait()

    def zb(g, _):
        o = pl.multiple_of(g * 16, 16)
        c = b1_ref[pl.ds(o, 16)]
        r = b2_ref[pl.ds(o, 16)]
        v = b3_ref[pl.ds(o, 16)]
        pc = plsc.load_gather(pred_ref, [c])
        plsc.addupdate_scatter(acc_ref, [r], v * pc)
        return 0
    lax.fori_loop(0, _GZ, zb, 0)
    merge_to(res_ref, jnp.add)

    def lpb(gi, v):
        o = pl.multiple_of(gi * 16, 16)
        r = res_ref[pl.ds(o, 16)] - bsl_ref[pl.ds(o, 16)]
        return v + out_sl_ref[pl.ds(o, 16)] * r * r
    lp_vec = lax.fori_loop(0, _GS, lpb, zero16)

    def ldb(gi, v):
        o = pl.multiple_of(gi * 16, 16)
        p = pred_ref[pl.ds(base + o, 16)]
        t = tsl_ref[pl.ds(o, 16)]
        d = p - t
        return v + d * d
    ld_vec = lax.fori_loop(0, _GS, ldb, zero16)

    vec16_ref[...] = ld_vec
    vecb16_ref[...] = lp_vec
    plsc.subcore_barrier()
    pltpu.sync_copy(vec16_ref, sh_arena.at[pl.ds(_VA + sid * 16, 16)])
    pltpu.sync_copy(vecb16_ref, sh_arena.at[pl.ds(_VB + sid * 16, 16)])
    plsc.subcore_barrier()

    @pl.when(sid == 0)
    def _():
        pltpu.sync_copy(sh_arena.at[pl.ds(_VA, 16 * _NT)], m16_ref)
        pltpu.sync_copy(sh_arena.at[pl.ds(_VB, 16 * _NT)], mb16_ref)
        sa = m16_ref[pl.ds(0, 16)]
        sb = mb16_ref[pl.ds(0, 16)]
        for j in range(1, _NT):
            sa = sa + m16_ref[pl.ds(j * 16, 16)]
            sb = sb + mb16_ref[pl.ds(j * 16, 16)]
        total = (jnp.sum(sa) + jnp.sum(sb)) * (1.0 / _N)
        outv_ref[...] = jnp.full((16,), total, jnp.float32)
        pltpu.sync_copy(outv_ref, out_hbm)


@jax.jit
def _sc_loss(pred_p, target_p, featsf, colind, vals, rowidx, b_p, edge_index):
    mesh = plsc.VectorSubcoreMesh(core_axis_name="c", subcore_axis_name="s",
                                  num_cores=1, num_subcores=_NT)
    f32 = jnp.float32
    run = pl.kernel(
        _kernel_body,
        out_type=jax.ShapeDtypeStruct((16,), f32),
        mesh=mesh,
        compiler_params=pltpu.CompilerParams(needs_layout_passes=False),
        scratch_types=[
            pltpu.VMEM((_NP,), f32),
            pltpu.VMEM((_NP,), f32),
            pltpu.VMEM((_NP,), f32),
            pltpu.VMEM((_EPT,), jnp.int32),
            pltpu.VMEM((_EPT,), jnp.int32),
            pltpu.VMEM((_ZPT,), f32),
            pltpu.VMEM((_NT, _SL), f32),
            pltpu.VMEM((16 * _NT,), f32),
            pltpu.VMEM((16 * _NT,), f32),
            pltpu.VMEM((16,), f32),
            pltpu.VMEM((16,), f32),
            pltpu.VMEM((_SL,), f32),
            pltpu.VMEM((_SL,), f32),
            pltpu.VMEM((_SL,), f32),
            pltpu.VMEM((_SL,), f32),
            pltpu.VMEM((_SL,), f32),
            pltpu.VMEM((_SL,), f32),
            pltpu.VMEM((16,), f32),
            pltpu.SemaphoreType.DMA,
            pltpu.VMEM_SHARED(((_NT + 1) * _NP + 32 * _NT,), f32),
        ],
    )
    return run(pred_p, target_p, featsf, colind, vals, rowidx, b_p, edge_index)


def kernel(pred, target, feats, A_row_ptr, A_col_ind, A_vals, A_row_idx, b,
           edge_index, epoch):
    del A_row_ptr, epoch
    padn = _NP - _N
    pred_p = jnp.pad(pred, (0, padn))
    target_p = jnp.pad(target, (0, padn))
    b_p = jnp.pad(b, (0, padn))
    featsf = feats.reshape(-1)
    out = _sc_loss(pred_p, target_p, featsf, A_col_ind, A_vals, A_row_idx,
                   b_p, edge_index.reshape(-1))
    return out[0]
